# baseline (speedup 1.0000x reference)
import jax
import jax.numpy as jnp
from jax.experimental import pallas as pl
from jax.experimental.pallas import tpu as pltpu

_H = 32
_CIN = 3
_K = 5
_C1, _C2 = 6, 16
_OH1 = _H - _K + 1
_P1 = _OH1 // 2
_OH2 = _P1 - _K + 1
_P2 = _OH2 // 2
_FC1, _FC2, _FC3 = 120, 84, 10
_TB = 512
_S1W = _P1 * _C1
_S2W = _P2 * _C2


def _prep_conv1(conv1_w):
    e = jnp.stack([jnp.eye(_H, _OH1, k=-dx, dtype=jnp.float32)
                   for dx in range(_K)])
    full = jnp.einsum('xjw,oiyx->iyjwo', e, conv1_w)
    full = full.reshape(_CIN, _K, _H, _P1, 2, _C1)
    full = full.transpose(1, 0, 2, 4, 3, 5)
    full = full.reshape(_K * _CIN * _H, 2, _S1W)
    full = jnp.pad(full, ((0, 0), (0, 0), (0, 128 - _S1W)))
    return full.reshape(_K * _CIN * _H, 256)


def _prep_conv2(conv2_w):
    e = jnp.stack([jnp.eye(_P1, _OH2, k=-dx, dtype=jnp.float32)
                   for dx in range(_K)])
    full = jnp.einsum('xjw,oiyx->yjiwo', e, conv2_w)
    full = full.reshape(_K, _S1W, _P2, 2, _C2)
    full = full.transpose(0, 1, 3, 2, 4)
    full = full.reshape(_K, _S1W, 2, _S2W)
    full = jnp.pad(full, ((0, 0), (0, 0), (0, 0), (0, 128 - _S2W)))
    return full.reshape(_K * _S1W, 256)


def _lenet_kernel(x_ref, w1_ref, b1_ref, w2_ref, b2_ref,
                  wf1_ref, bf1_ref, wf2_ref, bf2_ref, wf3_ref, bf3_ref,
                  o_ref, s1_ref, s2_ref):
    f32 = jnp.float32
    bf16 = jnp.bfloat16
    b1 = b1_ref[...]
    for i in range(_P1):
        m = None
        for t in range(2):
            oh = 2 * i + t
            base = oh * _CIN * _H
            r = jnp.dot(x_ref[:, base:base + _K * _CIN * _H], w1_ref[...],
                        preferred_element_type=f32)
            r = jnp.maximum(r + b1, 0.0)
            mm = jnp.maximum(r[:, :128], r[:, 128:])
            m = mm if m is None else jnp.maximum(m, mm)
        s1_ref[:, i * _S1W:(i + 1) * _S1W] = m[:, :_S1W].astype(bf16)

    b2 = b2_ref[...]
    for i2 in range(_P2):
        m = None
        for t in range(2):
            oh2 = 2 * i2 + t
            r = jnp.dot(s1_ref[:, oh2 * _S1W:(oh2 + _K) * _S1W], w2_ref[...],
                        preferred_element_type=f32)
            r = jnp.maximum(r + b2, 0.0)
            mm = jnp.maximum(r[:, :128], r[:, 128:])
            m = mm if m is None else jnp.maximum(m, mm)
        s2_ref[:, i2 * _S2W:(i2 + 1) * _S2W] = m[:, :_S2W].astype(bf16)

    h = jnp.dot(s2_ref[...], wf1_ref[...], preferred_element_type=f32)
    h = jnp.maximum(h + bf1_ref[...], 0.0).astype(bf16)
    h = jnp.dot(h, wf2_ref[...], preferred_element_type=f32)
    h = jnp.maximum(h + bf2_ref[...], 0.0).astype(bf16)
    o_ref[...] = (jnp.dot(h, wf3_ref[...], preferred_element_type=f32)
                  + bf3_ref[...])[:, :_FC3]


def kernel(conv1_w, conv1_b, conv2_w, conv2_b, fc1_w, fc1_b,
           fc2_w, fc2_b, fc3_w, fc3_b, x_nchw):
    B = x_nchw.shape[0]
    bf16 = jnp.bfloat16
    x = x_nchw.transpose(0, 2, 1, 3).reshape(B, _H * _CIN * _H)

    w1 = _prep_conv1(conv1_w)
    b1 = jnp.tile(jnp.pad(jnp.tile(conv1_b, _P1), (0, 128 - _S1W)),
                  2).reshape(1, 256)
    w2 = _prep_conv2(conv2_w).astype(bf16)
    b2 = jnp.tile(jnp.pad(jnp.tile(conv2_b, _P2), (0, 128 - _S2W)),
                  2).reshape(1, 256)
    wf1 = fc1_w.reshape(_FC1, _C2, _P2, _P2).transpose(2, 3, 1, 0)
    wf1 = jnp.pad(wf1.reshape(_P2 * _S2W, _FC1),
                  ((0, 0), (0, 128 - _FC1))).astype(bf16)
    bf1 = jnp.pad(fc1_b, (0, 128 - _FC1)).reshape(1, 128)
    wf2 = jnp.pad(fc2_w.T, ((0, 128 - _FC1), (0, 128 - _FC2))).astype(bf16)
    bf2 = jnp.pad(fc2_b, (0, 128 - _FC2)).reshape(1, 128)
    wf3 = jnp.pad(fc3_w.T, ((0, 128 - _FC2), (0, 128 - _FC3))).astype(bf16)
    bf3 = jnp.pad(fc3_b, (0, 128 - _FC3)).reshape(1, 128)

    out = pl.pallas_call(
        _lenet_kernel,
        out_shape=jax.ShapeDtypeStruct((B, _FC3), jnp.float32),
        grid=(pl.cdiv(B, _TB),),
        in_specs=[
            pl.BlockSpec((_TB, _CIN * _H * _H), lambda b: (b, 0)),
            pl.BlockSpec((_K * _CIN * _H, 256), lambda b: (0, 0)),
            pl.BlockSpec((1, 256), lambda b: (0, 0)),
            pl.BlockSpec((_K * _S1W, 256), lambda b: (0, 0)),
            pl.BlockSpec((1, 256), lambda b: (0, 0)),
            pl.BlockSpec((_P2 * _S2W, 128), lambda b: (0, 0)),
            pl.BlockSpec((1, 128), lambda b: (0, 0)),
            pl.BlockSpec((128, 128), lambda b: (0, 0)),
            pl.BlockSpec((1, 128), lambda b: (0, 0)),
            pl.BlockSpec((128, 128), lambda b: (0, 0)),
            pl.BlockSpec((1, 128), lambda b: (0, 0)),
        ],
        out_specs=pl.BlockSpec((_TB, _FC3), lambda b: (b, 0)),
        scratch_shapes=[
            pltpu.VMEM((_TB, _P1 * _S1W), jnp.bfloat16),
            pltpu.VMEM((_TB, _P2 * _S2W), jnp.bfloat16),
        ],
        compiler_params=pltpu.CompilerParams(
            dimension_semantics=("parallel",)),
    )(x, w1, b1, w2, b2, wf1, bf1, wf2, bf2, wf3, bf3)

    return out


# speedup vs baseline: 1.5025x; 1.5025x over previous
import jax
import jax.numpy as jnp
from jax.experimental import pallas as pl
from jax.experimental.pallas import tpu as pltpu

_H = 32
_CIN = 3
_K = 5
_C1, _C2 = 6, 16
_OH1 = _H - _K + 1
_P1 = _OH1 // 2
_OH2 = _P1 - _K + 1
_P2 = _OH2 // 2
_FC1, _FC2, _FC3 = 120, 84, 10
_TB = 512
_S1W = _P1 * _C1
_S2W = _P2 * _C2


def _prep_conv1(conv1_w):
    e = jnp.stack([jnp.eye(_H, _OH1, k=-dx, dtype=jnp.float32)
                   for dx in range(_K)])
    full = jnp.einsum('xjw,oiyx->iyjwo', e, conv1_w)
    full = full.reshape(_CIN, _K, _H, _P1, 2, _C1)
    full = full.transpose(1, 0, 2, 4, 3, 5)
    full = full.reshape(_K * _CIN * _H, 2, _S1W)
    full = jnp.pad(full, ((0, 0), (0, 0), (0, 128 - _S1W)))
    return full.reshape(_K * _CIN * _H, 256)


def _prep_conv2(conv2_w):
    e = jnp.stack([jnp.eye(_P1, _OH2, k=-dx, dtype=jnp.float32)
                   for dx in range(_K)])
    full = jnp.einsum('xjw,oiyx->yjiwo', e, conv2_w)
    full = full.reshape(_K, _S1W, _P2, 2, _C2)
    full = full.transpose(0, 1, 3, 2, 4)
    full = full.reshape(_K, _S1W, 2, _S2W)
    full = jnp.pad(full, ((0, 0), (0, 0), (0, 0), (0, 128 - _S2W)))
    return full.reshape(_K * _S1W, 256)


def _lenet_kernel(x_ref, w1_ref, b1_ref, w2_ref, b2_ref,
                  wf1_ref, bf1_ref, wf2_ref, bf2_ref, wf3_ref, bf3_ref,
                  o_ref, s1_ref, s2_ref):
    f32 = jnp.float32
    bf16 = jnp.bfloat16
    b1 = b1_ref[...]
    for i in range(_P1):
        m = None
        for t in range(2):
            oh = 2 * i + t
            base = oh * _CIN * _H
            r = jnp.dot(x_ref[:, base:base + _K * _CIN * _H], w1_ref[...],
                        preferred_element_type=f32)
            r = jnp.maximum(r + b1, 0.0)
            mm = jnp.maximum(r[:, :128], r[:, 128:])
            m = mm if m is None else jnp.maximum(m, mm)
        s1_ref[:, i * _S1W:(i + 1) * _S1W] = m[:, :_S1W].astype(bf16)

    b2 = b2_ref[...]
    for i2 in range(_P2):
        m = None
        for t in range(2):
            oh2 = 2 * i2 + t
            r = jnp.dot(s1_ref[:, oh2 * _S1W:(oh2 + _K) * _S1W], w2_ref[...],
                        preferred_element_type=f32)
            r = jnp.maximum(r + b2, 0.0)
            mm = jnp.maximum(r[:, :128], r[:, 128:])
            m = mm if m is None else jnp.maximum(m, mm)
        s2_ref[:, i2 * _S2W:(i2 + 1) * _S2W] = m[:, :_S2W].astype(bf16)

    h = jnp.dot(s2_ref[...], wf1_ref[...], preferred_element_type=f32)
    h = jnp.maximum(h + bf1_ref[...], 0.0).astype(bf16)
    h = jnp.dot(h, wf2_ref[...], preferred_element_type=f32)
    h = jnp.maximum(h + bf2_ref[...], 0.0).astype(bf16)
    o_ref[...] = (jnp.dot(h, wf3_ref[...], preferred_element_type=f32)
                  + bf3_ref[...])[:, :_FC3]


def kernel(conv1_w, conv1_b, conv2_w, conv2_b, fc1_w, fc1_b,
           fc2_w, fc2_b, fc3_w, fc3_b, x_nchw):
    B = x_nchw.shape[0]
    bf16 = jnp.bfloat16
    x = x_nchw.transpose(0, 2, 1, 3).reshape(B, _H * _CIN * _H).astype(bf16)

    w1 = _prep_conv1(conv1_w).astype(bf16)
    b1 = jnp.tile(jnp.pad(jnp.tile(conv1_b, _P1), (0, 128 - _S1W)),
                  2).reshape(1, 256)
    w2 = _prep_conv2(conv2_w).astype(bf16)
    b2 = jnp.tile(jnp.pad(jnp.tile(conv2_b, _P2), (0, 128 - _S2W)),
                  2).reshape(1, 256)
    wf1 = fc1_w.reshape(_FC1, _C2, _P2, _P2).transpose(2, 3, 1, 0)
    wf1 = jnp.pad(wf1.reshape(_P2 * _S2W, _FC1),
                  ((0, 0), (0, 128 - _FC1))).astype(bf16)
    bf1 = jnp.pad(fc1_b, (0, 128 - _FC1)).reshape(1, 128)
    wf2 = jnp.pad(fc2_w.T, ((0, 128 - _FC1), (0, 128 - _FC2))).astype(bf16)
    bf2 = jnp.pad(fc2_b, (0, 128 - _FC2)).reshape(1, 128)
    wf3 = jnp.pad(fc3_w.T, ((0, 128 - _FC2), (0, 128 - _FC3))).astype(bf16)
    bf3 = jnp.pad(fc3_b, (0, 128 - _FC3)).reshape(1, 128)

    out = pl.pallas_call(
        _lenet_kernel,
        out_shape=jax.ShapeDtypeStruct((B, _FC3), jnp.float32),
        grid=(pl.cdiv(B, _TB),),
        in_specs=[
            pl.BlockSpec((_TB, _CIN * _H * _H), lambda b: (b, 0)),
            pl.BlockSpec((_K * _CIN * _H, 256), lambda b: (0, 0)),
            pl.BlockSpec((1, 256), lambda b: (0, 0)),
            pl.BlockSpec((_K * _S1W, 256), lambda b: (0, 0)),
            pl.BlockSpec((1, 256), lambda b: (0, 0)),
            pl.BlockSpec((_P2 * _S2W, 128), lambda b: (0, 0)),
            pl.BlockSpec((1, 128), lambda b: (0, 0)),
            pl.BlockSpec((128, 128), lambda b: (0, 0)),
            pl.BlockSpec((1, 128), lambda b: (0, 0)),
            pl.BlockSpec((128, 128), lambda b: (0, 0)),
            pl.BlockSpec((1, 128), lambda b: (0, 0)),
        ],
        out_specs=pl.BlockSpec((_TB, _FC3), lambda b: (b, 0)),
        scratch_shapes=[
            pltpu.VMEM((_TB, _P1 * _S1W), jnp.bfloat16),
            pltpu.VMEM((_TB, _P2 * _S2W), jnp.bfloat16),
        ],
        compiler_params=pltpu.CompilerParams(
            dimension_semantics=("parallel",)),
    )(x, w1, b1, w2, b2, wf1, bf1, wf2, bf2, wf3, bf3)

    return out
